# baseline (speedup 1.0000x reference)
import functools

import jax
import jax.numpy as jnp
from jax import lax
from jax.experimental import pallas as pl
from jax.experimental.pallas import tpu as pltpu
from jax.experimental.pallas import tpu_sc as plsc

_NC = 2
_NS = 16
_LANES = 16


def _sc_build_tables(peX, peY):
    sqn, dh = peX.shape
    D = 2 * dh
    nw = _NS
    rows_per_w = (2 * sqn) // nw
    mesh = plsc.VectorSubcoreMesh(
        core_axis_name="c", subcore_axis_name="s",
        num_cores=1, num_subcores=_NS,
    )

    @functools.partial(
        pl.kernel,
        out_type=[
            jax.ShapeDtypeStruct((sqn, D), jnp.float32),
            jax.ShapeDtypeStruct((sqn, D), jnp.float32),
        ],
        mesh=mesh,
        scratch_types=(
            [pltpu.VMEM((dh,), jnp.float32) for _ in range(rows_per_w)]
            + [pltpu.VMEM((D,), jnp.float32) for _ in range(rows_per_w)]
            + [pltpu.SemaphoreType.DMA, pltpu.SemaphoreType.DMA]
        ),
        compiler_params=pltpu.CompilerParams(needs_layout_passes=False),
    )
    def build(peX_hbm, peY_hbm, peXi_hbm, peYi_hbm, *scratch):
        srcs = scratch[:rows_per_w]
        rows = scratch[rows_per_w:2 * rows_per_w]
        s_in, s_out = scratch[2 * rows_per_w:]
        wid = lax.axis_index("s")
        zero = jnp.zeros((_LANES,), jnp.float32)
        lane = lax.iota(jnp.int32, _LANES)
        parity = lane % 2
        half = rows_per_w // 2
        tabs = ((peX_hbm, peXi_hbm, 0), (peY_hbm, peYi_hbm, 1))
        hin = []
        for t, (src_hbm, dst_hbm, off) in enumerate(tabs):
            for j in range(half):
                r = wid * half + j
                hin.append(pltpu.async_copy(
                    src_hbm.at[r], srcs[t * half + j], s_in))
        for h in hin:
            h.wait()
        hout = []
        for t, (src_hbm, dst_hbm, off) in enumerate(tabs):
            for j in range(half):
                r = wid * half + j
                src_v = srcs[t * half + j]
                row_v = rows[t * half + j]

                def chunk(k, carry, src_v=src_v, row_v=row_v, off=off):
                    idx = (k * _LANES + lane) // 2
                    v = plsc.load_gather(src_v, [idx])
                    v = jnp.where(parity == off, v, zero)
                    row_v[pl.ds(k * _LANES, _LANES)] = v
                    return carry

                lax.fori_loop(0, D // _LANES, chunk, 0)
                hout.append(pltpu.async_copy(row_v, dst_hbm.at[r], s_out))
        for h in hout:
            h.wait()

    return build(peX, peY)


def _add_body(x_ref, pex_ref, pey_ref, o_ref):
    o_ref[...] = (
        x_ref[...]
        + pex_ref[0][None, None, :, :]
        + pey_ref[...][None, None, :, :]
    )


def kernel(x, peX, peY):
    B, N, D = x.shape
    sqn = peX.shape[0]
    peXi, peYi = _sc_build_tables(peX, peY)
    xr = x.reshape(B, sqn, sqn, D)
    out = pl.pallas_call(
        _add_body,
        grid=(sqn,),
        in_specs=[
            pl.BlockSpec((B, 1, sqn, D), lambda g: (0, g, 0, 0)),
            pl.BlockSpec((1, 1, D), lambda g: (g, 0, 0)),
            pl.BlockSpec((sqn, D), lambda g: (0, 0)),
        ],
        out_specs=pl.BlockSpec((B, 1, sqn, D), lambda g: (0, g, 0, 0)),
        out_shape=jax.ShapeDtypeStruct((B, sqn, sqn, D), x.dtype),
    )(xr, peXi.reshape(sqn, 1, D), peYi)
    return out.reshape(B, N, D)


# speedup vs baseline: 1.0094x; 1.0094x over previous
import functools

import jax
import jax.numpy as jnp
from jax import lax
from jax.experimental import pallas as pl
from jax.experimental.pallas import tpu as pltpu
from jax.experimental.pallas import tpu_sc as plsc

_NC = 2
_NS = 16
_LANES = 16


def _sc_build_tables(peX, peY):
    sqn, dh = peX.shape
    D = 2 * dh
    nw = _NC * _NS
    rows_per_w = (2 * sqn) // nw
    mesh = plsc.VectorSubcoreMesh(
        core_axis_name="c", subcore_axis_name="s",
        num_cores=_NC, num_subcores=_NS,
    )

    @functools.partial(
        pl.kernel,
        out_type=[
            jax.ShapeDtypeStruct((sqn, D), jnp.float32),
            jax.ShapeDtypeStruct((sqn, D), jnp.float32),
        ],
        mesh=mesh,
        scratch_types=(
            [pltpu.VMEM((dh,), jnp.float32) for _ in range(rows_per_w)]
            + [pltpu.VMEM((D,), jnp.float32) for _ in range(rows_per_w)]
            + [pltpu.SemaphoreType.DMA, pltpu.SemaphoreType.DMA]
        ),
        compiler_params=pltpu.CompilerParams(needs_layout_passes=False),
    )
    def build(peX_hbm, peY_hbm, peXi_hbm, peYi_hbm, *scratch):
        srcs = scratch[:rows_per_w]
        rows = scratch[rows_per_w:2 * rows_per_w]
        s_in, s_out = scratch[2 * rows_per_w:]
        wid = lax.axis_index("s") * _NC + lax.axis_index("c")
        zero = jnp.zeros((_LANES,), jnp.float32)
        lane = lax.iota(jnp.int32, _LANES)
        parity = lane % 2
        half = rows_per_w // 2
        tabs = ((peX_hbm, peXi_hbm, 0), (peY_hbm, peYi_hbm, 1))
        hin = []
        for t, (src_hbm, dst_hbm, off) in enumerate(tabs):
            for j in range(half):
                r = wid * half + j
                hin.append(pltpu.async_copy(
                    src_hbm.at[r], srcs[t * half + j], s_in))
        for h in hin:
            h.wait()
        hout = []
        for t, (src_hbm, dst_hbm, off) in enumerate(tabs):
            for j in range(half):
                r = wid * half + j
                src_v = srcs[t * half + j]
                row_v = rows[t * half + j]

                def chunk(k, carry, src_v=src_v, row_v=row_v, off=off):
                    idx = (k * _LANES + lane) // 2
                    v = plsc.load_gather(src_v, [idx])
                    v = jnp.where(parity == off, v, zero)
                    row_v[pl.ds(k * _LANES, _LANES)] = v
                    return carry

                lax.fori_loop(0, D // _LANES, chunk, 0)
                hout.append(pltpu.async_copy(row_v, dst_hbm.at[r], s_out))
        for h in hout:
            h.wait()

    return build(peX, peY)


def _add_body(x_ref, pex_ref, pey_ref, o_ref):
    o_ref[...] = (
        x_ref[...]
        + pex_ref[0][None, None, :, :]
        + pey_ref[...][None, None, :, :]
    )


def kernel(x, peX, peY):
    B, N, D = x.shape
    sqn = peX.shape[0]
    peXi, peYi = _sc_build_tables(peX, peY)
    xr = x.reshape(B, sqn, sqn, D)
    out = pl.pallas_call(
        _add_body,
        grid=(sqn,),
        in_specs=[
            pl.BlockSpec((B, 1, sqn, D), lambda g: (0, g, 0, 0)),
            pl.BlockSpec((1, 1, D), lambda g: (g, 0, 0)),
            pl.BlockSpec((sqn, D), lambda g: (0, 0)),
        ],
        out_specs=pl.BlockSpec((B, 1, sqn, D), lambda g: (0, g, 0, 0)),
        out_shape=jax.ShapeDtypeStruct((B, sqn, sqn, D), x.dtype),
    )(xr, peXi.reshape(sqn, 1, D), peYi)
    return out.reshape(B, N, D)
